# baseline (speedup 1.0000x reference)
import functools

import jax
import jax.numpy as jnp
from jax.experimental import pallas as pl
from jax.experimental.pallas import tpu as pltpu

_N = 1024
_D = 32
_R = 8


def _triplet_kernel(emb_full_ref, emb_blk_ref, labels_ref, labels_col_ref,
                    diff_ref, pos_ref, neg_ref):
    i = pl.program_id(0)
    e_full = emb_full_ref[:]
    e_blk = emb_blk_ref[:]
    diff = e_full[None, :, :] - e_blk[:, None, :]
    diff_ref[:] = diff
    d2 = jnp.sum(diff * diff, axis=-1)
    dist = jnp.sqrt(d2 + 1e-12)

    lbl = labels_ref[0, :]
    lbl_blk = labels_col_ref[pl.ds(i * _R, _R), 0]
    same = lbl_blk[:, None] == lbl[None, :]
    col = jax.lax.broadcasted_iota(jnp.int32, (_R, _N), 1)
    row = i * _R + jax.lax.broadcasted_iota(jnp.int32, (_R, _N), 0)
    not_eye = col != row

    neg_inf = jnp.float32(-jnp.inf)
    pos_inf = jnp.float32(jnp.inf)
    pos_d = jnp.where(same & not_eye, dist, neg_inf)
    neg_d = jnp.where(same, pos_inf, dist)

    pos_max = jnp.max(pos_d, axis=1, keepdims=True)
    pos_idx = jnp.min(jnp.where(pos_d == pos_max, col, _N), axis=1)
    neg_min = jnp.min(neg_d, axis=1, keepdims=True)
    neg_idx = jnp.min(jnp.where(neg_d == neg_min, col, _N), axis=1)

    pos_ref[pl.ds(i * _R, _R), 0] = pos_idx.astype(jnp.int32)
    neg_ref[pl.ds(i * _R, _R), 0] = neg_idx.astype(jnp.int32)


@jax.jit
def kernel(embeddings, labels):
    nb = _N // _R
    labels2d = labels.reshape(1, _N)
    grid_spec = pl.GridSpec(
        grid=(nb,),
        in_specs=[
            pl.BlockSpec((_N, _D), lambda i: (0, 0)),
            pl.BlockSpec((_R, _D), lambda i: (i, 0)),
            pl.BlockSpec((1, _N), lambda i: (0, 0)),
            pl.BlockSpec((_N, 1), lambda i: (0, 0)),
        ],
        out_specs=[
            pl.BlockSpec((_R, _N, _D), lambda i: (i, 0, 0)),
            pl.BlockSpec((_N, 1), lambda i: (0, 0)),
            pl.BlockSpec((_N, 1), lambda i: (0, 0)),
        ],
    )
    pair_diff, pos2d, neg2d = pl.pallas_call(
        _triplet_kernel,
        grid_spec=grid_spec,
        out_shape=[
            jax.ShapeDtypeStruct((_N, _N, _D), jnp.float32),
            jax.ShapeDtypeStruct((_N, 1), jnp.int32),
            jax.ShapeDtypeStruct((_N, 1), jnp.int32),
        ],
    )(embeddings, embeddings, labels2d, labels2d.reshape(_N, 1))
    return pair_diff, pos2d.reshape(_N), neg2d.reshape(_N)
